# baseline (speedup 1.0000x reference)
import functools

import jax
import jax.numpy as jnp
from jax import lax
from jax.experimental import pallas as pl
from jax.experimental.pallas import tpu as pltpu
from jax.experimental.pallas import tpu_sc as plsc

E = 320000
D = 128
NC = 2
NS = 16
NW = NC * NS
EPW = E // NW
G = 80
NCHUNK = EPW // G
L = 16

_mesh = plsc.VectorSubcoreMesh(core_axis_name="c", subcore_axis_name="s")


@functools.partial(
    pl.kernel,
    mesh=_mesh,
    out_type=jax.ShapeDtypeStruct((E,), jnp.float32),
    compiler_params=pltpu.CompilerParams(needs_layout_passes=False),
    scratch_types=[
        pltpu.VMEM((EPW,), jnp.int32),
        pltpu.VMEM((EPW,), jnp.int32),
        pltpu.VMEM((G, D), jnp.float32),
        pltpu.VMEM((G, D), jnp.float32),
        pltpu.VMEM((G,), jnp.float32),
        pltpu.SemaphoreType.DMA,
        pltpu.SemaphoreType.DMA,
    ],
)
def _ip_kernel(x_hbm, row_hbm, col_hbm, out_hbm,
               rows_v, cols_v, src_v, dst_v, out_v, sem_s, sem_d):
    wid = lax.axis_index("s") * NC + lax.axis_index("c")
    wbase = wid * EPW
    pltpu.sync_copy(row_hbm.at[pl.ds(wbase, EPW)], rows_v)
    pltpu.sync_copy(col_hbm.at[pl.ds(wbase, EPW)], cols_v)

    def chunk(g, carry):
        csl = pl.ds(g * G, G)
        cs = pltpu.async_copy(x_hbm.at[rows_v.at[csl]], src_v, sem_s)
        cd = pltpu.async_copy(x_hbm.at[cols_v.at[csl]], dst_v, sem_d)
        cs.wait()
        cd.wait()
        for e0 in range(0, G, L):
            rows = jnp.arange(e0, e0 + L, dtype=jnp.int32)

            def dstep(dd, acc):
                cidx = jnp.full((L,), dd, dtype=jnp.int32)
                sv = plsc.load_gather(src_v, [rows, cidx])
                dv = plsc.load_gather(dst_v, [rows, cidx])
                return acc + sv * dv

            acc = lax.fori_loop(0, D, dstep, jnp.zeros((L,), jnp.float32),
                                unroll=16)
            out_v[pl.ds(e0, L)] = acc
        pltpu.sync_copy(out_v, out_hbm.at[pl.ds(wbase + g * G, G)])
        return carry

    lax.fori_loop(0, NCHUNK, chunk, 0)


def kernel(x, edge_label_index):
    eli = edge_label_index.astype(jnp.int32)
    out = _ip_kernel(x, eli[0], eli[1])
    return out[:, None]


# speedup vs baseline: 1.1345x; 1.1345x over previous
import functools

import jax
import jax.numpy as jnp
from jax import lax
from jax.experimental import pallas as pl
from jax.experimental.pallas import tpu as pltpu
from jax.experimental.pallas import tpu_sc as plsc

E = 320000
D = 128
NC = 2
NS = 16
NW = NC * NS
EPW = E // NW
G = 80
NCHUNK = EPW // G
NBUF = 4
L = 16

_mesh = plsc.VectorSubcoreMesh(core_axis_name="c", subcore_axis_name="s")


@functools.partial(
    pl.kernel,
    mesh=_mesh,
    out_type=jax.ShapeDtypeStruct((E,), jnp.float32),
    compiler_params=pltpu.CompilerParams(needs_layout_passes=False),
    scratch_types=[
        pltpu.VMEM((EPW,), jnp.int32),
        pltpu.VMEM((EPW,), jnp.int32),
        pltpu.VMEM((NBUF, G, D), jnp.float32),
        pltpu.VMEM((NBUF, G, D), jnp.float32),
        pltpu.VMEM((EPW,), jnp.float32),
        pltpu.SemaphoreType.DMA((NBUF,)),
        pltpu.SemaphoreType.DMA((NBUF,)),
    ],
)
def _ip_kernel(x_hbm, row_hbm, col_hbm, out_hbm,
               rows_v, cols_v, src_v, dst_v, out_v, sem_s, sem_d):
    wid = lax.axis_index("s") * NC + lax.axis_index("c")
    wbase = wid * EPW
    pltpu.sync_copy(row_hbm.at[pl.ds(wbase, EPW)], rows_v)
    pltpu.sync_copy(col_hbm.at[pl.ds(wbase, EPW)], cols_v)

    def fire(g, b):
        csl = pl.ds(g * G, G)
        pltpu.async_copy(x_hbm.at[rows_v.at[csl]], src_v.at[b], sem_s.at[b])
        pltpu.async_copy(x_hbm.at[cols_v.at[csl]], dst_v.at[b], sem_d.at[b])

    def wait(g, b):
        csl = pl.ds(g * G, G)
        pltpu.make_async_copy(
            x_hbm.at[rows_v.at[csl]], src_v.at[b], sem_s.at[b]).wait()
        pltpu.make_async_copy(
            x_hbm.at[cols_v.at[csl]], dst_v.at[b], sem_d.at[b]).wait()

    def compute(g, b):
        for e0 in range(0, G, L):
            rows = jnp.arange(e0, e0 + L, dtype=jnp.int32)

            def dstep(dd, acc):
                cidx = jnp.full((L,), dd, dtype=jnp.int32)
                sv = plsc.load_gather(src_v.at[b], [rows, cidx])
                dv = plsc.load_gather(dst_v.at[b], [rows, cidx])
                return acc + sv * dv

            acc = lax.fori_loop(0, D, dstep, jnp.zeros((L,), jnp.float32),
                                unroll=16)
            out_v[pl.ds(g * G + e0, L)] = acc

    for b in range(NBUF - 1):
        fire(b, b)

    def step(i, carry):
        for b in range(NBUF):
            g = i * NBUF + b

            @pl.when(g + NBUF - 1 < NCHUNK)
            def _():
                fire(g + NBUF - 1, (b + NBUF - 1) % NBUF)

            wait(g, b)
            compute(g, b)
        return carry

    lax.fori_loop(0, NCHUNK // NBUF, step, 0)
    for g in range(NBUF * (NCHUNK // NBUF), NCHUNK):
        b = g % NBUF
        wait(g, b)
        compute(g, b)

    pltpu.sync_copy(out_v, out_hbm.at[pl.ds(wbase, EPW)])


def kernel(x, edge_label_index):
    eli = edge_label_index.astype(jnp.int32)
    out = _ip_kernel(x, eli[0], eli[1])
    return out[:, None]


# speedup vs baseline: 6.6973x; 5.9035x over previous
import functools

import jax
import jax.numpy as jnp
from jax import lax
from jax.experimental import pallas as pl
from jax.experimental.pallas import tpu as pltpu
from jax.experimental.pallas import tpu_sc as plsc

E = 320000
D = 128
NC = 2
NS = 16
NW = NC * NS
EPW = E // NW
G = 80
NCHUNK = EPW // G
NBUF = 4
L = 16

_mesh = plsc.VectorSubcoreMesh(core_axis_name="c", subcore_axis_name="s")


@functools.partial(
    pl.kernel,
    mesh=_mesh,
    out_type=jax.ShapeDtypeStruct((E,), jnp.float32),
    compiler_params=pltpu.CompilerParams(needs_layout_passes=False),
    scratch_types=[
        pltpu.VMEM((EPW,), jnp.int32),
        pltpu.VMEM((EPW,), jnp.int32),
        pltpu.VMEM((NBUF, G, D), jnp.float32),
        pltpu.VMEM((NBUF, G, D), jnp.float32),
        pltpu.VMEM((EPW,), jnp.float32),
        pltpu.VMEM((L * (L + 1),), jnp.float32),
        pltpu.SemaphoreType.DMA((NBUF,)),
        pltpu.SemaphoreType.DMA((NBUF,)),
    ],
)
def _ip_kernel(x_hbm, row_hbm, col_hbm, out_hbm,
               rows_v, cols_v, src_v, dst_v, out_v, tbuf, sem_s, sem_d):
    wid = lax.axis_index("s") * NC + lax.axis_index("c")
    wbase = wid * EPW
    pltpu.sync_copy(row_hbm.at[pl.ds(wbase, EPW)], rows_v)
    pltpu.sync_copy(col_hbm.at[pl.ds(wbase, EPW)], cols_v)

    def fire(g, b):
        csl = pl.ds(g * G, G)
        pltpu.async_copy(x_hbm.at[rows_v.at[csl]], src_v.at[b], sem_s.at[b])
        pltpu.async_copy(x_hbm.at[cols_v.at[csl]], dst_v.at[b], sem_d.at[b])

    def wait(g, b):
        csl = pl.ds(g * G, G)
        pltpu.make_async_copy(
            x_hbm.at[rows_v.at[csl]], src_v.at[b], sem_s.at[b]).wait()
        pltpu.make_async_copy(
            x_hbm.at[cols_v.at[csl]], dst_v.at[b], sem_d.at[b]).wait()

    def compute(g, b):
        def group(g2, carry):
            ebase = g2 * L
            for e in range(L):
                row = ebase + e
                acc = src_v[b, row, pl.ds(0, L)] * dst_v[b, row, pl.ds(0, L)]
                for k in range(1, D // L):
                    acc += (src_v[b, row, pl.ds(k * L, L)]
                            * dst_v[b, row, pl.ds(k * L, L)])
                tbuf[pl.ds(e * (L + 1), L)] = acc
            cols = jnp.arange(0, L * (L + 1), L + 1, dtype=jnp.int32)
            w = plsc.load_gather(tbuf, [cols])
            for k in range(1, L):
                w += plsc.load_gather(tbuf, [cols + k])
            out_v[pl.ds(g * G + ebase, L)] = w
            return carry

        lax.fori_loop(0, G // L, group, 0)

    for b in range(NBUF - 1):
        fire(b, b)

    def step(i, carry):
        for b in range(NBUF):
            g = i * NBUF + b

            @pl.when(g + NBUF - 1 < NCHUNK)
            def _():
                fire(g + NBUF - 1, (b + NBUF - 1) % NBUF)

            wait(g, b)
            compute(g, b)
        return carry

    lax.fori_loop(0, NCHUNK // NBUF, step, 0)
    for g in range(NBUF * (NCHUNK // NBUF), NCHUNK):
        b = g % NBUF
        wait(g, b)
        compute(g, b)

    pltpu.sync_copy(out_v, out_hbm.at[pl.ds(wbase, EPW)])


def kernel(x, edge_label_index):
    eli = edge_label_index.astype(jnp.int32)
    out = _ip_kernel(x, eli[0], eli[1])
    return out[:, None]
